# Initial kernel scaffold; baseline (speedup 1.0000x reference)
import jax
import jax.numpy as jnp
from jax.experimental import pallas as pl


def kernel(feature_torch, edge_torch, i, j, W0, b0, W1, b1, fc1_W, fc1_b, fc2_W, fc2_b):
    raise NotImplementedError("write your pallas kernel here")


# speedup vs baseline: 59.5893x; 59.5893x over previous
import functools

import jax
import jax.numpy as jnp
from jax import lax
from jax.experimental import pallas as pl
from jax.experimental.pallas import tpu as pltpu
from jax.experimental.pallas import tpu_sc as plsc

N = 100000
NP = 102400
NPT = NP // 16
F32 = jnp.float32
I32 = jnp.int32

_MESH = plsc.VectorSubcoreMesh(core_axis_name="c", subcore_axis_name="s")


def _worker_rows(cid, sid, rows):
    w = cid * 16 + sid
    q = rows // 32
    rem = rows % 32
    base = w * q + jnp.minimum(w, rem)
    n = q + (w < rem).astype(I32)
    return base, n


def _sc_deg(rows, edge, zeros, degp, dst_st, ones_v, deg_sh):
    cid = lax.axis_index("c")
    sid = lax.axis_index("s")
    for k in range(8):
        ones_v[pl.ds(k * 16, 16)] = jnp.ones((16,), F32)

    @pl.when(sid == 0)
    def _():
        pltpu.sync_copy(zeros, deg_sh)

    plsc.subcore_barrier()
    base, n = _worker_rows(cid, sid, rows)

    def body(it, carry):
        r = base + it
        pltpu.sync_copy(edge.at[1, pl.ds(r * 128, 128)], dst_st.at[0])
        pltpu.sync_copy(ones_v, deg_sh.at[dst_st.at[0]], add=True)
        return carry

    lax.fori_loop(0, n, body, 0)
    plsc.subcore_barrier()

    @pl.when(sid == 0)
    def _():
        pltpu.sync_copy(deg_sh, degp.at[cid])


def _tc_rsqrt(degp_ref, out_ref):
    out_ref[...] = lax.rsqrt(jnp.maximum(degp_ref[0] + degp_ref[1], 1.0))


def _sc_main(rows, edge, dn, x0, x1, ij, zeros,
             u0p, u1p, cip, cjp,
             st, y0v, y1v, miv, mjv, ijv, dnb, xb, yb,
             y0_sh, y1_sh, u0_sh, u1_sh, ci_sh, cj_sh):
    cid = lax.axis_index("c")
    sid = lax.axis_index("s")
    pltpu.sync_copy(ij, ijv)

    nb = sid * NPT
    pltpu.sync_copy(dn.at[pl.ds(nb, NPT)], dnb)
    pltpu.sync_copy(x0.at[pl.ds(nb, NPT)], xb)

    def ymul(k, carry):
        yb[pl.ds(k * 16, 16)] = xb[pl.ds(k * 16, 16)] * dnb[pl.ds(k * 16, 16)]
        return carry

    lax.fori_loop(0, NPT // 16, ymul, 0)
    pltpu.sync_copy(yb, y0_sh.at[pl.ds(nb, NPT)])
    pltpu.sync_copy(x1.at[pl.ds(nb, NPT)], xb)
    lax.fori_loop(0, NPT // 16, ymul, 0)
    pltpu.sync_copy(yb, y1_sh.at[pl.ds(nb, NPT)])

    for which, acc in ((0, u0_sh), (1, u1_sh), (2, ci_sh), (3, cj_sh)):
        @pl.when(sid == which)
        def _(acc=acc):
            pltpu.sync_copy(zeros, acc)

    plsc.subcore_barrier()
    base, n = _worker_rows(cid, sid, rows)

    def body(it, carry):
        r = base + it
        pltpu.sync_copy(edge.at[0, pl.ds(r * 128, 128)], st.at[0])
        pltpu.sync_copy(edge.at[1, pl.ds(r * 128, 128)], st.at[1])
        pltpu.sync_copy(y0_sh.at[st.at[0]], y0v)
        pltpu.sync_copy(y1_sh.at[st.at[0]], y1v)
        one = jnp.ones((16,), F32)
        zero = jnp.zeros((16,), F32)
        for k in range(8):
            dv = st[1, pl.ds(k * 16, 16)]
            miv[pl.ds(k * 16, 16)] = jnp.where(dv == ijv[0, :], one, zero)
            mjv[pl.ds(k * 16, 16)] = jnp.where(dv == ijv[1, :], one, zero)
        pltpu.sync_copy(y0v, u0_sh.at[st.at[1]], add=True)
        pltpu.sync_copy(y1v, u1_sh.at[st.at[1]], add=True)
        pltpu.sync_copy(miv, ci_sh.at[st.at[0]], add=True)
        pltpu.sync_copy(mjv, cj_sh.at[st.at[0]], add=True)
        return carry

    lax.fori_loop(0, n, body, 0)
    plsc.subcore_barrier()
    for which, acc, out in ((0, u0_sh, u0p), (1, u1_sh, u1p),
                            (2, ci_sh, cip), (3, cj_sh, cjp)):
        @pl.when(sid == which)
        def _(acc=acc, out=out):
            pltpu.sync_copy(acc, out.at[cid])


def _tc_finish(dn_ref, u0_ref, u1_ref, ci_ref, cj_ref,
               dnij_ref, W0_ref, b0_ref, W1_ref, b1_ref,
               fc1W_ref, fc1b_ref, fc2W_ref, fc2b_ref, out_ref):
    d = dn_ref[...]
    t0 = d * (u0_ref[0] + u0_ref[1])
    t1 = d * (u1_ref[0] + u1_ref[1])
    wi = d * (ci_ref[0] + ci_ref[1])
    wj = d * (cj_ref[0] + cj_ref[1])
    P = []
    Q = []
    for f in range(16):
        h = jnp.maximum(t0 * W0_ref[0, f] + t1 * W0_ref[1, f] + b0_ref[0, f],
                        0.0)
        P.append(jnp.sum(wi * h))
        Q.append(jnp.sum(wj * h))
    dni = dnij_ref[0, 0]
    dnj = dnij_ref[0, 1]
    embd = []
    for vals, dsc in ((P, dni), (Q, dnj)):
        for g in range(16):
            a = b1_ref[0, g]
            for f in range(16):
                a = a + dsc * vals[f] * W1_ref[f, g]
            embd.append(jnp.maximum(a, 0.0))
    res = []
    for c in range(2):
        a = fc2b_ref[0, c]
        for hh in range(40):
            r = fc1b_ref[0, hh]
            for k in range(32):
                r = r + embd[k] * fc1W_ref[k, hh]
            a = a + jnp.maximum(r, 0.0) * fc2W_ref[hh, c]
        res.append(a)
    ri = lax.broadcasted_iota(I32, (8, 128), 0)
    li = lax.broadcasted_iota(I32, (8, 128), 1)
    out = jnp.where((ri == 0) & (li == 0), res[0],
                    jnp.where((ri == 0) & (li == 1), res[1], 0.0))
    out_ref[...] = out


def kernel(feature_torch, edge_torch, i, j, W0, b0, W1, b1,
           fc1_W, fc1_b, fc2_W, fc2_b):
    E = edge_torch.shape[1]
    pad_e = (-E) % 128
    if pad_e:
        edge_torch = jnp.pad(edge_torch, ((0, 0), (0, pad_e)),
                             constant_values=N)
    rows = edge_torch.shape[1] // 128

    zeros = jnp.zeros((NP,), F32)
    x0 = jnp.pad(feature_torch[:, 0], (0, NP - N))
    x1 = jnp.pad(feature_torch[:, 1], (0, NP - N))
    ij = jnp.stack([jnp.full((16,), i, I32), jnp.full((16,), j, I32)])

    degp = pl.kernel(
        functools.partial(_sc_deg, rows),
        out_type=jax.ShapeDtypeStruct((2, NP), F32),
        mesh=_MESH,
        scratch_types=[
            pltpu.VMEM((1, 128), I32),
            pltpu.VMEM((128,), F32),
            pltpu.VMEM_SHARED((NP,), F32),
        ],
    )(edge_torch, zeros)

    dn2 = pl.pallas_call(
        _tc_rsqrt,
        out_shape=jax.ShapeDtypeStruct((800, 128), F32),
    )(degp.reshape(2, 800, 128))
    dnp = dn2.reshape(NP)

    u0p, u1p, cip, cjp = pl.kernel(
        functools.partial(_sc_main, rows),
        out_type=[jax.ShapeDtypeStruct((2, NP), F32)] * 4,
        mesh=_MESH,
        scratch_types=[
            pltpu.VMEM((2, 128), I32),
            pltpu.VMEM((128,), F32),
            pltpu.VMEM((128,), F32),
            pltpu.VMEM((128,), F32),
            pltpu.VMEM((128,), F32),
            pltpu.VMEM((2, 16), I32),
            pltpu.VMEM((NPT,), F32),
            pltpu.VMEM((NPT,), F32),
            pltpu.VMEM((NPT,), F32),
            pltpu.VMEM_SHARED((NP,), F32),
            pltpu.VMEM_SHARED((NP,), F32),
            pltpu.VMEM_SHARED((NP,), F32),
            pltpu.VMEM_SHARED((NP,), F32),
            pltpu.VMEM_SHARED((NP,), F32),
            pltpu.VMEM_SHARED((NP,), F32),
        ],
    )(edge_torch, dnp, x0, x1, ij, zeros)

    dnij = jnp.stack([dnp[i], dnp[j]]).reshape(1, 2)
    smem = pl.BlockSpec(memory_space=pltpu.SMEM)
    vmem = pl.BlockSpec(memory_space=pltpu.VMEM)
    out_pad = pl.pallas_call(
        _tc_finish,
        out_shape=jax.ShapeDtypeStruct((8, 128), F32),
        in_specs=[vmem] * 5 + [smem] * 9,
        out_specs=vmem,
    )(dn2,
      u0p.reshape(2, 800, 128), u1p.reshape(2, 800, 128),
      cip.reshape(2, 800, 128), cjp.reshape(2, 800, 128),
      dnij, W0, b0.reshape(1, 16), W1, b1.reshape(1, 16),
      fc1_W, fc1_b.reshape(1, 40), fc2_W, fc2_b.reshape(1, 2))
    return out_pad[0, :2]


# speedup vs baseline: 265.2720x; 4.4517x over previous
import functools

import jax
import jax.numpy as jnp
from jax import lax
from jax.experimental import pallas as pl
from jax.experimental.pallas import tpu as pltpu
from jax.experimental.pallas import tpu_sc as plsc

N = 100000
NP = 102400
NPT = NP // 16
F32 = jnp.float32
I32 = jnp.int32

_MESH = plsc.VectorSubcoreMesh(core_axis_name="c", subcore_axis_name="s")
CB = 16


def _worker_chunks(cid, sid, nchunks_total):
    w = cid * 16 + sid
    q = nchunks_total // 32
    rem = nchunks_total % 32
    base = w * q + jnp.minimum(w, rem)
    n = q + (w < rem).astype(I32)
    return w, base, n


def _sc_deg(rows, edge3, zeros, degp, dst_blk, ones_v, deg_sh, sem_s):
    cid = lax.axis_index("c")
    sid = lax.axis_index("s")
    for k in range(8):
        ones_v[pl.ds(k * 16, 16)] = jnp.ones((16,), F32)

    @pl.when(sid == 0)
    def _():
        pltpu.sync_copy(zeros, deg_sh)

    plsc.subcore_barrier()
    nct = rows // CB
    tail = rows % CB
    w, base, n = _worker_chunks(cid, sid, nct)

    def chunk(c, carry):
        r0 = (base + c) * CB
        pltpu.sync_copy(edge3.at[1, pl.ds(r0, CB)], dst_blk)

        def fire(r, cc):
            pltpu.async_copy(ones_v, deg_sh.at[dst_blk.at[r]], sem_s,
                             add=True)
            return cc

        lax.fori_loop(0, CB, fire, 0)

        def drain(r, cc):
            pltpu.make_async_copy(ones_v, deg_sh.at[dst_blk.at[r]],
                                  sem_s).wait()
            return cc

        lax.fori_loop(0, CB, drain, 0)
        return carry

    lax.fori_loop(0, n, chunk, 0)

    if tail:
        @pl.when(w == 31)
        def _():
            pltpu.sync_copy(edge3.at[1, pl.ds(nct * CB, tail)],
                            dst_blk.at[pl.ds(0, tail)])

            def trow(r, cc):
                pltpu.sync_copy(ones_v, deg_sh.at[dst_blk.at[r]], add=True)
                return cc

            lax.fori_loop(0, tail, trow, 0)

    plsc.subcore_barrier()

    @pl.when(sid == 0)
    def _():
        pltpu.sync_copy(deg_sh, degp.at[cid])


def _tc_rsqrt(degp_ref, out_ref):
    out_ref[...] = lax.rsqrt(jnp.maximum(degp_ref[0] + degp_ref[1], 1.0))


def _sc_main(rows, edge3, dn, x0, x1, ij, zeros,
             u0p, u1p, cip, cjp,
             src_blk, dst_blk, g0, g1, mi_blk, mj_blk, ijv, dnb, xb, yb,
             y0_sh, y1_sh, u0_sh, u1_sh, ci_sh, cj_sh, sem_g, sem_s):
    cid = lax.axis_index("c")
    sid = lax.axis_index("s")
    pltpu.sync_copy(ij, ijv)

    nb = sid * NPT
    pltpu.sync_copy(dn.at[pl.ds(nb, NPT)], dnb)
    pltpu.sync_copy(x0.at[pl.ds(nb, NPT)], xb)

    def ymul(k, carry):
        yb[pl.ds(k * 16, 16)] = xb[pl.ds(k * 16, 16)] * dnb[pl.ds(k * 16, 16)]
        return carry

    lax.fori_loop(0, NPT // 16, ymul, 0)
    pltpu.sync_copy(yb, y0_sh.at[pl.ds(nb, NPT)])
    pltpu.sync_copy(x1.at[pl.ds(nb, NPT)], xb)
    lax.fori_loop(0, NPT // 16, ymul, 0)
    pltpu.sync_copy(yb, y1_sh.at[pl.ds(nb, NPT)])

    for which, acc in ((0, u0_sh), (1, u1_sh), (2, ci_sh), (3, cj_sh)):
        @pl.when(sid == which)
        def _(acc=acc):
            pltpu.sync_copy(zeros, acc)

    plsc.subcore_barrier()
    nct = rows // CB
    tail = rows % CB
    w, base, n = _worker_chunks(cid, sid, nct)
    ivv = ijv[0, :]
    jvv = ijv[1, :]
    one = jnp.ones((16,), F32)
    zero = jnp.zeros((16,), F32)

    def masks_row(r):
        acc = zero
        for k in range(8):
            dv = dst_blk[r, pl.ds(k * 16, 16)]
            mi = jnp.where(dv == ivv, one, zero)
            mj = jnp.where(dv == jvv, one, zero)
            mi_blk[r, pl.ds(k * 16, 16)] = mi
            mj_blk[r, pl.ds(k * 16, 16)] = mj
            acc = acc + mi + mj
        return 0

    def chunk(c, carry):
        r0 = (base + c) * CB
        pltpu.sync_copy(edge3.at[0, pl.ds(r0, CB)], src_blk)
        pltpu.sync_copy(edge3.at[1, pl.ds(r0, CB)], dst_blk)

        def fire_g(r, cc):
            pltpu.async_copy(y0_sh.at[src_blk.at[r]], g0.at[r], sem_g)
            pltpu.async_copy(y1_sh.at[src_blk.at[r]], g1.at[r], sem_g)
            return cc

        lax.fori_loop(0, CB, fire_g, 0)

        def mrow(r, hacc):
            masks_row(r)
            return hacc

        lax.fori_loop(0, CB, mrow, 0)

        def drain_g(r, cc):
            pltpu.make_async_copy(y0_sh.at[src_blk.at[r]], g0.at[r],
                                  sem_g).wait()
            pltpu.make_async_copy(y1_sh.at[src_blk.at[r]], g1.at[r],
                                  sem_g).wait()
            return cc

        lax.fori_loop(0, CB, drain_g, 0)

        def fire_s(r, cc):
            pltpu.async_copy(g0.at[r], u0_sh.at[dst_blk.at[r]], sem_s,
                             add=True)
            pltpu.async_copy(g1.at[r], u1_sh.at[dst_blk.at[r]], sem_s,
                             add=True)
            return cc

        lax.fori_loop(0, CB, fire_s, 0)

        def fire_c(r, cc):
            pltpu.async_copy(mi_blk.at[r], ci_sh.at[src_blk.at[r]], sem_s,
                             add=True)
            pltpu.async_copy(mj_blk.at[r], cj_sh.at[src_blk.at[r]], sem_s,
                             add=True)
            return cc

        lax.fori_loop(0, CB, fire_c, 0)

        def drain_s(r, cc):
            pltpu.make_async_copy(g0.at[r], u0_sh.at[dst_blk.at[r]],
                                  sem_s).wait()
            pltpu.make_async_copy(g1.at[r], u1_sh.at[dst_blk.at[r]],
                                  sem_s).wait()
            pltpu.make_async_copy(mi_blk.at[r], ci_sh.at[src_blk.at[r]],
                                  sem_s).wait()
            pltpu.make_async_copy(mj_blk.at[r], cj_sh.at[src_blk.at[r]],
                                  sem_s).wait()
            return cc

        lax.fori_loop(0, CB, drain_s, 0)
        return carry

    lax.fori_loop(0, n, chunk, 0)

    if tail:
        @pl.when(w == 31)
        def _():
            pltpu.sync_copy(edge3.at[0, pl.ds(nct * CB, tail)],
                            src_blk.at[pl.ds(0, tail)])
            pltpu.sync_copy(edge3.at[1, pl.ds(nct * CB, tail)],
                            dst_blk.at[pl.ds(0, tail)])

            def trow(r, cc):
                pltpu.sync_copy(y0_sh.at[src_blk.at[r]], g0.at[r])
                pltpu.sync_copy(y1_sh.at[src_blk.at[r]], g1.at[r])
                masks_row(r)
                pltpu.sync_copy(g0.at[r], u0_sh.at[dst_blk.at[r]], add=True)
                pltpu.sync_copy(g1.at[r], u1_sh.at[dst_blk.at[r]], add=True)
                pltpu.sync_copy(mi_blk.at[r], ci_sh.at[src_blk.at[r]],
                                add=True)
                pltpu.sync_copy(mj_blk.at[r], cj_sh.at[src_blk.at[r]],
                                add=True)
                return cc

            lax.fori_loop(0, tail, trow, 0)

    plsc.subcore_barrier()
    for which, acc, out in ((0, u0_sh, u0p), (1, u1_sh, u1p),
                            (2, ci_sh, cip), (3, cj_sh, cjp)):
        @pl.when(sid == which)
        def _(acc=acc, out=out):
            pltpu.sync_copy(acc, out.at[cid])


def _tc_finish(dn_ref, u0_ref, u1_ref, ci_ref, cj_ref,
               dnij_ref, W0_ref, b0_ref, W1_ref, b1_ref,
               fc1W_ref, fc1b_ref, fc2W_ref, fc2b_ref, out_ref):
    d = dn_ref[...]
    t0 = d * (u0_ref[0] + u0_ref[1])
    t1 = d * (u1_ref[0] + u1_ref[1])
    wi = d * (ci_ref[0] + ci_ref[1])
    wj = d * (cj_ref[0] + cj_ref[1])
    P = []
    Q = []
    for f in range(16):
        h = jnp.maximum(t0 * W0_ref[0, f] + t1 * W0_ref[1, f] + b0_ref[0, f],
                        0.0)
        P.append(jnp.sum(wi * h))
        Q.append(jnp.sum(wj * h))
    dni = dnij_ref[0, 0]
    dnj = dnij_ref[0, 1]
    embd = []
    for vals, dsc in ((P, dni), (Q, dnj)):
        for g in range(16):
            a = b1_ref[0, g]
            for f in range(16):
                a = a + dsc * vals[f] * W1_ref[f, g]
            embd.append(jnp.maximum(a, 0.0))
    res = []
    for c in range(2):
        a = fc2b_ref[0, c]
        for hh in range(40):
            r = fc1b_ref[0, hh]
            for k in range(32):
                r = r + embd[k] * fc1W_ref[k, hh]
            a = a + jnp.maximum(r, 0.0) * fc2W_ref[hh, c]
        res.append(a)
    ri = lax.broadcasted_iota(I32, (8, 128), 0)
    li = lax.broadcasted_iota(I32, (8, 128), 1)
    out = jnp.where((ri == 0) & (li == 0), res[0],
                    jnp.where((ri == 0) & (li == 1), res[1], 0.0))
    out_ref[...] = out


def kernel(feature_torch, edge_torch, i, j, W0, b0, W1, b1,
           fc1_W, fc1_b, fc2_W, fc2_b):
    E = edge_torch.shape[1]
    pad_e = (-E) % 128
    if pad_e:
        edge_torch = jnp.pad(edge_torch, ((0, 0), (0, pad_e)),
                             constant_values=N)
    rows = edge_torch.shape[1] // 128
    edge3 = edge_torch.reshape(2, rows, 128)

    zeros = jnp.zeros((NP,), F32)
    x0 = jnp.pad(feature_torch[:, 0], (0, NP - N))
    x1 = jnp.pad(feature_torch[:, 1], (0, NP - N))
    ij = jnp.stack([jnp.full((16,), i, I32), jnp.full((16,), j, I32)])

    degp = pl.kernel(
        functools.partial(_sc_deg, rows),
        out_type=jax.ShapeDtypeStruct((2, NP), F32),
        mesh=_MESH,
        scratch_types=[
            pltpu.VMEM((CB, 128), I32),
            pltpu.VMEM((128,), F32),
            pltpu.VMEM_SHARED((NP,), F32),
            pltpu.SemaphoreType.DMA,
        ],
    )(edge3, zeros)

    dn2 = pl.pallas_call(
        _tc_rsqrt,
        out_shape=jax.ShapeDtypeStruct((800, 128), F32),
    )(degp.reshape(2, 800, 128))
    dnp = dn2.reshape(NP)

    u0p, u1p, cip, cjp = pl.kernel(
        functools.partial(_sc_main, rows),
        out_type=[jax.ShapeDtypeStruct((2, NP), F32)] * 4,
        mesh=_MESH,
        scratch_types=[
            pltpu.VMEM((CB, 128), I32),
            pltpu.VMEM((CB, 128), I32),
            pltpu.VMEM((CB, 128), F32),
            pltpu.VMEM((CB, 128), F32),
            pltpu.VMEM((CB, 128), F32),
            pltpu.VMEM((CB, 128), F32),
            pltpu.VMEM((2, 16), I32),
            pltpu.VMEM((NPT,), F32),
            pltpu.VMEM((NPT,), F32),
            pltpu.VMEM((NPT,), F32),
            pltpu.VMEM_SHARED((NP,), F32),
            pltpu.VMEM_SHARED((NP,), F32),
            pltpu.VMEM_SHARED((NP,), F32),
            pltpu.VMEM_SHARED((NP,), F32),
            pltpu.VMEM_SHARED((NP,), F32),
            pltpu.VMEM_SHARED((NP,), F32),
            pltpu.SemaphoreType.DMA,
            pltpu.SemaphoreType.DMA,
        ],
    )(edge3, dnp, x0, x1, ij, zeros)

    dnij = jnp.stack([dnp[i], dnp[j]]).reshape(1, 2)
    smem = pl.BlockSpec(memory_space=pltpu.SMEM)
    vmem = pl.BlockSpec(memory_space=pltpu.VMEM)
    out_pad = pl.pallas_call(
        _tc_finish,
        out_shape=jax.ShapeDtypeStruct((8, 128), F32),
        in_specs=[vmem] * 5 + [smem] * 9,
        out_specs=vmem,
    )(dn2,
      u0p.reshape(2, 800, 128), u1p.reshape(2, 800, 128),
      cip.reshape(2, 800, 128), cjp.reshape(2, 800, 128),
      dnij, W0, b0.reshape(1, 16), W1, b1.reshape(1, 16),
      fc1_W, fc1_b.reshape(1, 40), fc2_W, fc2_b.reshape(1, 2))
    return out_pad[0, :2]


# speedup vs baseline: 309.4521x; 1.1665x over previous
import functools

import jax
import jax.numpy as jnp
from jax import lax
from jax.experimental import pallas as pl
from jax.experimental.pallas import tpu as pltpu
from jax.experimental.pallas import tpu_sc as plsc

N = 100000
NP = 102400
NPT = NP // 16
F32 = jnp.float32
I32 = jnp.int32

_MESH = plsc.VectorSubcoreMesh(core_axis_name="c", subcore_axis_name="s")
CB = 16


def _worker_chunks(cid, sid, nchunks_total):
    w = cid * 16 + sid
    q = nchunks_total // 32
    rem = nchunks_total % 32
    base = w * q + jnp.minimum(w, rem)
    n = q + (w < rem).astype(I32)
    return w, base, n


def _sc_deg(rows, edge3, zeros, degp, dst_blk, ones_v, deg_sh, sem_s):
    cid = lax.axis_index("c")
    sid = lax.axis_index("s")
    for k in range(8):
        ones_v[pl.ds(k * 16, 16)] = jnp.ones((16,), F32)

    @pl.when(sid == 0)
    def _():
        pltpu.sync_copy(zeros, deg_sh)

    plsc.subcore_barrier()
    nct = rows // CB
    tail = rows % CB
    w, base, n = _worker_chunks(cid, sid, nct)

    def chunk(c, carry):
        r0 = (base + c) * CB
        pltpu.sync_copy(edge3.at[1, pl.ds(r0, CB)], dst_blk)

        def fire(r, cc):
            pltpu.async_copy(ones_v, deg_sh.at[dst_blk.at[r]], sem_s,
                             add=True)
            return cc

        lax.fori_loop(0, CB, fire, 0)

        def drain(r, cc):
            pltpu.make_async_copy(ones_v, deg_sh.at[dst_blk.at[r]],
                                  sem_s).wait()
            return cc

        lax.fori_loop(0, CB, drain, 0)
        return carry

    lax.fori_loop(0, n, chunk, 0)

    if tail:
        @pl.when(w == 31)
        def _():
            pltpu.sync_copy(edge3.at[1, pl.ds(nct * CB, tail)],
                            dst_blk.at[pl.ds(0, tail)])

            def trow(r, cc):
                pltpu.sync_copy(ones_v, deg_sh.at[dst_blk.at[r]], add=True)
                return cc

            lax.fori_loop(0, tail, trow, 0)

    plsc.subcore_barrier()

    @pl.when(sid == 0)
    def _():
        pltpu.sync_copy(deg_sh, degp.at[cid])


def _tc_rsqrt(degp_ref, out_ref):
    out_ref[...] = lax.rsqrt(jnp.maximum(degp_ref[0] + degp_ref[1], 1.0))


def _sc_main(rows, edge3, dn, x0, x1, ij, zeros,
             u0p, u1p, cip, cjp,
             src_blk, dst_blk, g0, g1, mi_blk, mj_blk, ijv, dnb, xb, yb,
             fvmem, fsmem,
             y0_sh, y1_sh, u0_sh, u1_sh, ci_sh, cj_sh, flag_sh,
             sem_g, sem_s):
    cid = lax.axis_index("c")
    sid = lax.axis_index("s")
    pltpu.sync_copy(ij, ijv)

    nb = sid * NPT
    pltpu.sync_copy(dn.at[pl.ds(nb, NPT)], dnb)
    pltpu.sync_copy(x0.at[pl.ds(nb, NPT)], xb)

    def ymul(k, carry):
        yb[pl.ds(k * 16, 16)] = xb[pl.ds(k * 16, 16)] * dnb[pl.ds(k * 16, 16)]
        return carry

    lax.fori_loop(0, NPT // 16, ymul, 0)
    pltpu.sync_copy(yb, y0_sh.at[pl.ds(nb, NPT)])
    pltpu.sync_copy(x1.at[pl.ds(nb, NPT)], xb)
    lax.fori_loop(0, NPT // 16, ymul, 0)
    pltpu.sync_copy(yb, y1_sh.at[pl.ds(nb, NPT)])

    for which, acc in ((0, u0_sh), (1, u1_sh), (2, ci_sh), (3, cj_sh)):
        @pl.when(sid == which)
        def _(acc=acc):
            pltpu.sync_copy(zeros, acc)

    plsc.subcore_barrier()
    nct = rows // CB
    tail = rows % CB
    w, base, n = _worker_chunks(cid, sid, nct)
    ivv = ijv[0, :]
    jvv = ijv[1, :]
    one = jnp.ones((16,), F32)
    zero = jnp.zeros((16,), F32)
    lanes = lax.iota(I32, 16)

    def masks_row(r):
        acc = zero
        for k in range(8):
            dv = dst_blk[r, pl.ds(k * 16, 16)]
            mi = jnp.where(dv == ivv, one, zero)
            mj = jnp.where(dv == jvv, one, zero)
            mi_blk[r, pl.ds(k * 16, 16)] = mi
            mj_blk[r, pl.ds(k * 16, 16)] = mj
            acc = acc + mi + mj
        for d in (1, 2, 4, 8):
            acc = acc + acc.at[lanes ^ d].get(mode="promise_in_bounds",
                                              unique_indices=True)
        return acc

    def chunk(c, carry):
        r0 = (base + c) * CB
        pltpu.sync_copy(edge3.at[0, pl.ds(r0, CB)], src_blk)
        pltpu.sync_copy(edge3.at[1, pl.ds(r0, CB)], dst_blk)

        def fire_g(r, cc):
            pltpu.async_copy(y0_sh.at[src_blk.at[r]], g0.at[r], sem_g)
            pltpu.async_copy(y1_sh.at[src_blk.at[r]], g1.at[r], sem_g)
            return cc

        lax.fori_loop(0, CB, fire_g, 0)

        def mrow(r, flagvec):
            acc = masks_row(r)
            return jnp.where(lanes == r, acc, flagvec)

        flagvec = lax.fori_loop(0, CB, mrow, zero)
        fvmem[...] = flagvec
        pltpu.sync_copy(fvmem, flag_sh.at[sid])
        pltpu.sync_copy(flag_sh.at[sid], fsmem)

        def drain_g(r, cc):
            pltpu.make_async_copy(y0_sh.at[src_blk.at[r]], g0.at[r],
                                  sem_g).wait()
            pltpu.make_async_copy(y1_sh.at[src_blk.at[r]], g1.at[r],
                                  sem_g).wait()
            return cc

        lax.fori_loop(0, CB, drain_g, 0)

        def fire_s(r, cc):
            pltpu.async_copy(g0.at[r], u0_sh.at[dst_blk.at[r]], sem_s,
                             add=True)
            pltpu.async_copy(g1.at[r], u1_sh.at[dst_blk.at[r]], sem_s,
                             add=True)
            return cc

        lax.fori_loop(0, CB, fire_s, 0)

        def fire_c(r, cc):
            @pl.when(fsmem[r] != 0.0)
            def _():
                pltpu.sync_copy(mi_blk.at[r], ci_sh.at[src_blk.at[r]],
                                add=True)
                pltpu.sync_copy(mj_blk.at[r], cj_sh.at[src_blk.at[r]],
                                add=True)

            return cc

        lax.fori_loop(0, CB, fire_c, 0)

        def drain_s(r, cc):
            pltpu.make_async_copy(g0.at[r], u0_sh.at[dst_blk.at[r]],
                                  sem_s).wait()
            pltpu.make_async_copy(g1.at[r], u1_sh.at[dst_blk.at[r]],
                                  sem_s).wait()
            return cc

        lax.fori_loop(0, CB, drain_s, 0)
        return carry

    lax.fori_loop(0, n, chunk, 0)

    if tail:
        @pl.when(w == 31)
        def _():
            pltpu.sync_copy(edge3.at[0, pl.ds(nct * CB, tail)],
                            src_blk.at[pl.ds(0, tail)])
            pltpu.sync_copy(edge3.at[1, pl.ds(nct * CB, tail)],
                            dst_blk.at[pl.ds(0, tail)])

            def trow(r, cc):
                pltpu.sync_copy(y0_sh.at[src_blk.at[r]], g0.at[r])
                pltpu.sync_copy(y1_sh.at[src_blk.at[r]], g1.at[r])
                masks_row(r)
                pltpu.sync_copy(g0.at[r], u0_sh.at[dst_blk.at[r]], add=True)
                pltpu.sync_copy(g1.at[r], u1_sh.at[dst_blk.at[r]], add=True)
                pltpu.sync_copy(mi_blk.at[r], ci_sh.at[src_blk.at[r]],
                                add=True)
                pltpu.sync_copy(mj_blk.at[r], cj_sh.at[src_blk.at[r]],
                                add=True)
                return cc

            lax.fori_loop(0, tail, trow, 0)

    plsc.subcore_barrier()
    for which, acc, out in ((0, u0_sh, u0p), (1, u1_sh, u1p),
                            (2, ci_sh, cip), (3, cj_sh, cjp)):
        @pl.when(sid == which)
        def _(acc=acc, out=out):
            pltpu.sync_copy(acc, out.at[cid])


def _tc_finish(dn_ref, u0_ref, u1_ref, ci_ref, cj_ref,
               dnij_ref, W0_ref, b0_ref, W1_ref, b1_ref,
               fc1W_ref, fc1b_ref, fc2W_ref, fc2b_ref, out_ref):
    d = dn_ref[...]
    t0 = d * (u0_ref[0] + u0_ref[1])
    t1 = d * (u1_ref[0] + u1_ref[1])
    wi = d * (ci_ref[0] + ci_ref[1])
    wj = d * (cj_ref[0] + cj_ref[1])
    P = []
    Q = []
    for f in range(16):
        h = jnp.maximum(t0 * W0_ref[0, f] + t1 * W0_ref[1, f] + b0_ref[0, f],
                        0.0)
        P.append(jnp.sum(wi * h))
        Q.append(jnp.sum(wj * h))
    dni = dnij_ref[0, 0]
    dnj = dnij_ref[0, 1]
    embd = []
    for vals, dsc in ((P, dni), (Q, dnj)):
        for g in range(16):
            a = b1_ref[0, g]
            for f in range(16):
                a = a + dsc * vals[f] * W1_ref[f, g]
            embd.append(jnp.maximum(a, 0.0))
    res = []
    for c in range(2):
        a = fc2b_ref[0, c]
        for hh in range(40):
            r = fc1b_ref[0, hh]
            for k in range(32):
                r = r + embd[k] * fc1W_ref[k, hh]
            a = a + jnp.maximum(r, 0.0) * fc2W_ref[hh, c]
        res.append(a)
    ri = lax.broadcasted_iota(I32, (8, 128), 0)
    li = lax.broadcasted_iota(I32, (8, 128), 1)
    out = jnp.where((ri == 0) & (li == 0), res[0],
                    jnp.where((ri == 0) & (li == 1), res[1], 0.0))
    out_ref[...] = out


def kernel(feature_torch, edge_torch, i, j, W0, b0, W1, b1,
           fc1_W, fc1_b, fc2_W, fc2_b):
    E = edge_torch.shape[1]
    pad_e = (-E) % 128
    if pad_e:
        edge_torch = jnp.pad(edge_torch, ((0, 0), (0, pad_e)),
                             constant_values=N)
    rows = edge_torch.shape[1] // 128
    edge3 = edge_torch.reshape(2, rows, 128)

    zeros = jnp.zeros((NP,), F32)
    x0 = jnp.pad(feature_torch[:, 0], (0, NP - N))
    x1 = jnp.pad(feature_torch[:, 1], (0, NP - N))
    ij = jnp.stack([jnp.full((16,), i, I32), jnp.full((16,), j, I32)])

    degp = pl.kernel(
        functools.partial(_sc_deg, rows),
        out_type=jax.ShapeDtypeStruct((2, NP), F32),
        mesh=_MESH,
        scratch_types=[
            pltpu.VMEM((CB, 128), I32),
            pltpu.VMEM((128,), F32),
            pltpu.VMEM_SHARED((NP,), F32),
            pltpu.SemaphoreType.DMA,
        ],
    )(edge3, zeros)

    dn2 = pl.pallas_call(
        _tc_rsqrt,
        out_shape=jax.ShapeDtypeStruct((800, 128), F32),
    )(degp.reshape(2, 800, 128))
    dnp = dn2.reshape(NP)

    u0p, u1p, cip, cjp = pl.kernel(
        functools.partial(_sc_main, rows),
        out_type=[jax.ShapeDtypeStruct((2, NP), F32)] * 4,
        mesh=_MESH,
        scratch_types=[
            pltpu.VMEM((CB, 128), I32),
            pltpu.VMEM((CB, 128), I32),
            pltpu.VMEM((CB, 128), F32),
            pltpu.VMEM((CB, 128), F32),
            pltpu.VMEM((CB, 128), F32),
            pltpu.VMEM((CB, 128), F32),
            pltpu.VMEM((2, 16), I32),
            pltpu.VMEM((NPT,), F32),
            pltpu.VMEM((NPT,), F32),
            pltpu.VMEM((NPT,), F32),
            pltpu.VMEM((16,), F32),
            pltpu.SMEM((16,), F32),
            pltpu.VMEM_SHARED((NP,), F32),
            pltpu.VMEM_SHARED((NP,), F32),
            pltpu.VMEM_SHARED((NP,), F32),
            pltpu.VMEM_SHARED((NP,), F32),
            pltpu.VMEM_SHARED((NP,), F32),
            pltpu.VMEM_SHARED((NP,), F32),
            pltpu.VMEM_SHARED((16, 16), F32),
            pltpu.SemaphoreType.DMA,
            pltpu.SemaphoreType.DMA,
        ],
    )(edge3, dnp, x0, x1, ij, zeros)

    dnij = jnp.stack([dnp[i], dnp[j]]).reshape(1, 2)
    smem = pl.BlockSpec(memory_space=pltpu.SMEM)
    vmem = pl.BlockSpec(memory_space=pltpu.VMEM)
    out_pad = pl.pallas_call(
        _tc_finish,
        out_shape=jax.ShapeDtypeStruct((8, 128), F32),
        in_specs=[vmem] * 5 + [smem] * 9,
        out_specs=vmem,
    )(dn2,
      u0p.reshape(2, 800, 128), u1p.reshape(2, 800, 128),
      cip.reshape(2, 800, 128), cjp.reshape(2, 800, 128),
      dnij, W0, b0.reshape(1, 16), W1, b1.reshape(1, 16),
      fc1_W, fc1_b.reshape(1, 40), fc2_W, fc2_b.reshape(1, 2))
    return out_pad[0, :2]
